# speedup vs baseline: 5.1848x; 5.1848x over previous
import functools

import jax
import jax.numpy as jnp
from jax.experimental import pallas as pl
from jax.experimental.pallas import tpu as pltpu


def _seq_body(x_ref, wxp_ref, wh_ref, ww_ref, h0_ref, hw0_ref, b_ref, bias_ref,
              h_all_ref, rv_all_ref, tape_ref, hw_s, xp_s, *, tblk, scale):
    i = pl.program_id(0)
    B, D = hw0_ref.shape
    BN = tape_ref.shape[0]

    @pl.when(i == 0)
    def _init():
        tape_ref[...] = h0_ref[...]
        hw_s[...] = hw0_ref[...]

    xblk = x_ref[...].reshape(tblk * B, D)
    xp_s[...] = jnp.dot(xblk, wxp_ref[...],
                        preferred_element_type=jnp.float32).reshape(tblk, B, D)

    bias = bias_ref[...]
    ones8 = jnp.ones((B, 8), jnp.float32)

    def step(t, h_work):
        xp = xp_s[t]
        H = tape_ref[...]
        s_r = jax.lax.dot_general(
            h_work, H, (((1,), (1,)), ((), ())),
            preferred_element_type=jnp.float32) * scale + bias
        e_r = jnp.exp(s_r - jnp.max(s_r, axis=-1, keepdims=True))
        a_r = e_r / jnp.sum(e_r, axis=-1, keepdims=True)
        read_val = jnp.dot(a_r, H, preferred_element_type=jnp.float32)
        h_lin = jnp.dot(h_work, wh_ref[...], preferred_element_type=jnp.float32)
        h_new = jnp.tanh(xp + h_lin + read_val + b_ref[...])
        wv = jnp.dot(h_new, ww_ref[...], preferred_element_type=jnp.float32)
        s_w = jax.lax.dot_general(
            wv, H, (((1,), (1,)), ((), ())),
            preferred_element_type=jnp.float32) * scale + bias
        e_w = jnp.exp(s_w - jnp.max(s_w, axis=-1, keepdims=True))
        a_w = e_w / jnp.sum(e_w, axis=-1, keepdims=True)
        wa = jax.lax.dot_general(
            a_w, ones8, (((0,), (0,)), ((), ())),
            preferred_element_type=jnp.float32)[:, :1]
        wv_exp = jnp.broadcast_to(
            wv[:, None, :], (B, BN // B, D)).reshape(BN, D)
        tape_ref[...] = H + wa * (wv_exp - H)
        h_all_ref[t] = h_new
        rv_all_ref[t] = read_val
        return h_new

    hw_s[...] = jax.lax.fori_loop(0, tblk, step, hw_s[...])


def _gate_body(x_ref, h_ref, rv_ref, wz_ref, wg_ref, o_ref, *, tblk):
    B, D = x_ref.shape[1], x_ref.shape[2]
    m = tblk * B
    xb = x_ref[...].reshape(m, D)
    z = jnp.dot(xb, wz_ref[...], preferred_element_type=jnp.float32)
    h = h_ref[...].reshape(m, D)
    rv = rv_ref[...].reshape(m, D)
    gin = jnp.concatenate([z, rv, h], axis=1)
    g = jnp.dot(gin, wg_ref[...], preferred_element_type=jnp.float32)
    o_ref[...] = (h * g * jax.nn.sigmoid(g)).reshape(tblk, B, D)


def _build(interpret=False):
    TBLK = 32
    GBLK = 32

    def forward(x, h_tape_init, h_work_init, W_h, W_xz, b_h, W_write, W_gate):
        B, T, D = x.shape
        N = h_tape_init.shape[1]
        BN = B * N
        scale = 1.0 / float(D) ** 0.5

        x_t = jnp.swapaxes(x, 0, 1)
        Wxz_T = W_xz.T
        Wxp = Wxz_T[:, :D]
        Wz = Wxz_T[:, D:]
        H0 = h_tape_init.reshape(BN, D)
        b2 = b_h.reshape(1, D)
        cols = jax.lax.broadcasted_iota(jnp.int32, (B, BN), 1) // N
        rows = jax.lax.broadcasted_iota(jnp.int32, (B, BN), 0)
        bias = jnp.where(cols == rows, 0.0, -1e9).astype(jnp.float32)

        h_all, rv_all, tape_f = pl.pallas_call(
            functools.partial(_seq_body, tblk=TBLK, scale=scale),
            grid=(T // TBLK,),
            in_specs=[
                pl.BlockSpec((TBLK, B, D), lambda i: (i, 0, 0)),
                pl.BlockSpec((D, D), lambda i: (0, 0)),
                pl.BlockSpec((D, D), lambda i: (0, 0)),
                pl.BlockSpec((D, D), lambda i: (0, 0)),
                pl.BlockSpec((BN, D), lambda i: (0, 0)),
                pl.BlockSpec((B, D), lambda i: (0, 0)),
                pl.BlockSpec((1, D), lambda i: (0, 0)),
                pl.BlockSpec((B, BN), lambda i: (0, 0)),
            ],
            out_specs=[
                pl.BlockSpec((TBLK, B, D), lambda i: (i, 0, 0)),
                pl.BlockSpec((TBLK, B, D), lambda i: (i, 0, 0)),
                pl.BlockSpec((BN, D), lambda i: (0, 0)),
            ],
            out_shape=[
                jax.ShapeDtypeStruct((T, B, D), jnp.float32),
                jax.ShapeDtypeStruct((T, B, D), jnp.float32),
                jax.ShapeDtypeStruct((BN, D), jnp.float32),
            ],
            scratch_shapes=[
                pltpu.VMEM((B, D), jnp.float32),
                pltpu.VMEM((TBLK, B, D), jnp.float32),
            ],
            compiler_params=pltpu.CompilerParams(
                dimension_semantics=("arbitrary",),
                vmem_limit_bytes=56 * 1024 * 1024,
            ),
            name="elman_seq",
            interpret=interpret,
        )(x_t, Wxp, W_h.T, W_write.T, H0, h_work_init, b2, bias)

        out_t = pl.pallas_call(
            functools.partial(_gate_body, tblk=GBLK),
            grid=(T // GBLK,),
            in_specs=[
                pl.BlockSpec((GBLK, B, D), lambda i: (i, 0, 0)),
                pl.BlockSpec((GBLK, B, D), lambda i: (i, 0, 0)),
                pl.BlockSpec((GBLK, B, D), lambda i: (i, 0, 0)),
                pl.BlockSpec((D, D), lambda i: (0, 0)),
                pl.BlockSpec((3 * D, D), lambda i: (0, 0)),
            ],
            out_specs=pl.BlockSpec((GBLK, B, D), lambda i: (i, 0, 0)),
            out_shape=jax.ShapeDtypeStruct((T, B, D), jnp.float32),
            compiler_params=pltpu.CompilerParams(
                dimension_semantics=("parallel",),
                vmem_limit_bytes=56 * 1024 * 1024,
            ),
            name="elman_gate",
            interpret=interpret,
        )(x_t, h_all, rv_all, Wz, W_gate.T)

        return jnp.swapaxes(out_t, 0, 1), tape_f.reshape(B, N, D)

    return forward


_forward = _build()


def kernel(x, h_tape_init, h_work_init, W_h, W_xz, b_h, W_write, W_gate):
    return _forward(x, h_tape_init, h_work_init, W_h, W_xz, b_h,
                    W_write, W_gate)


# speedup vs baseline: 5.1995x; 1.0028x over previous
import functools

import jax
import jax.numpy as jnp
from jax.experimental import pallas as pl
from jax.experimental.pallas import tpu as pltpu


def _seq_body(x_ref, wxp_ref, wh_ref, ww_ref, h0_ref, hw0_ref, b_ref, bias_ref,
              h_all_ref, rv_all_ref, tape_ref, hw_s, xp_s, tape16_s,
              *, tblk, scale):
    i = pl.program_id(0)
    B, D = hw0_ref.shape
    BN = tape_ref.shape[0]
    bf16 = jnp.bfloat16

    @pl.when(i == 0)
    def _init():
        tape_ref[...] = h0_ref[...]
        tape16_s[...] = h0_ref[...].astype(bf16)
        hw_s[...] = hw0_ref[...]

    xblk = x_ref[...].reshape(tblk * B, D).astype(bf16)
    xp_s[...] = jnp.dot(xblk, wxp_ref[...],
                        preferred_element_type=jnp.float32).reshape(tblk, B, D)

    bias = bias_ref[...]
    ones8 = jnp.ones((B, 8), bf16)

    def step(t, h_work):
        xp = xp_s[t]
        H16 = tape16_s[...]
        hw16 = h_work.astype(bf16)
        s_r = jax.lax.dot_general(
            hw16, H16, (((1,), (1,)), ((), ())),
            preferred_element_type=jnp.float32) * scale + bias
        e_r = jnp.exp(s_r - jnp.max(s_r, axis=-1, keepdims=True))
        a_r = (e_r / jnp.sum(e_r, axis=-1, keepdims=True)).astype(bf16)
        read_val = jnp.dot(a_r, H16, preferred_element_type=jnp.float32)
        h_lin = jnp.dot(hw16, wh_ref[...], preferred_element_type=jnp.float32)
        h_new = jnp.tanh(xp + h_lin + read_val + b_ref[...])
        h16 = h_new.astype(bf16)
        wv = jnp.dot(h16, ww_ref[...], preferred_element_type=jnp.float32)
        wv16 = wv.astype(bf16)
        s_w = jax.lax.dot_general(
            wv16, H16, (((1,), (1,)), ((), ())),
            preferred_element_type=jnp.float32) * scale + bias
        e_w = jnp.exp(s_w - jnp.max(s_w, axis=-1, keepdims=True))
        a_w = (e_w / jnp.sum(e_w, axis=-1, keepdims=True)).astype(bf16)
        wa = jax.lax.dot_general(
            a_w, ones8, (((0,), (0,)), ((), ())),
            preferred_element_type=jnp.float32)[:, :1]
        wv_exp = jnp.broadcast_to(
            wv[:, None, :], (B, BN // B, D)).reshape(BN, D)
        H_new = tape_ref[...] + wa * (wv_exp - tape_ref[...])
        tape_ref[...] = H_new
        tape16_s[...] = H_new.astype(bf16)
        h_all_ref[t] = h_new
        rv_all_ref[t] = read_val
        return h_new

    hw_s[...] = jax.lax.fori_loop(0, tblk, step, hw_s[...])


def _gate_body(x_ref, h_ref, rv_ref, wz_ref, wg_ref, o_ref, *, tblk):
    B, D = x_ref.shape[1], x_ref.shape[2]
    m = tblk * B
    bf16 = jnp.bfloat16
    xb = x_ref[...].reshape(m, D).astype(bf16)
    z = jnp.dot(xb, wz_ref[...], preferred_element_type=jnp.float32)
    h = h_ref[...].reshape(m, D)
    rv = rv_ref[...].reshape(m, D)
    gin = jnp.concatenate(
        [z.astype(bf16), rv.astype(bf16), h.astype(bf16)], axis=1)
    g = jnp.dot(gin, wg_ref[...], preferred_element_type=jnp.float32)
    o_ref[...] = (h * g * jax.nn.sigmoid(g)).reshape(tblk, B, D)


def _build(interpret=False):
    TBLK = 32
    GBLK = 32

    def forward(x, h_tape_init, h_work_init, W_h, W_xz, b_h, W_write, W_gate):
        B, T, D = x.shape
        N = h_tape_init.shape[1]
        BN = B * N
        scale = 1.0 / float(D) ** 0.5

        x_t = jnp.swapaxes(x, 0, 1)
        Wxz_T = W_xz.T.astype(jnp.bfloat16)
        Wxp = Wxz_T[:, :D]
        Wz = Wxz_T[:, D:]
        H0 = h_tape_init.reshape(BN, D)
        b2 = b_h.reshape(1, D)
        cols = jax.lax.broadcasted_iota(jnp.int32, (B, BN), 1) // N
        rows = jax.lax.broadcasted_iota(jnp.int32, (B, BN), 0)
        bias = jnp.where(cols == rows, 0.0, -1e9).astype(jnp.float32)

        h_all, rv_all, tape_f = pl.pallas_call(
            functools.partial(_seq_body, tblk=TBLK, scale=scale),
            grid=(T // TBLK,),
            in_specs=[
                pl.BlockSpec((TBLK, B, D), lambda i: (i, 0, 0)),
                pl.BlockSpec((D, D), lambda i: (0, 0)),
                pl.BlockSpec((D, D), lambda i: (0, 0)),
                pl.BlockSpec((D, D), lambda i: (0, 0)),
                pl.BlockSpec((BN, D), lambda i: (0, 0)),
                pl.BlockSpec((B, D), lambda i: (0, 0)),
                pl.BlockSpec((1, D), lambda i: (0, 0)),
                pl.BlockSpec((B, BN), lambda i: (0, 0)),
            ],
            out_specs=[
                pl.BlockSpec((TBLK, B, D), lambda i: (i, 0, 0)),
                pl.BlockSpec((TBLK, B, D), lambda i: (i, 0, 0)),
                pl.BlockSpec((BN, D), lambda i: (0, 0)),
            ],
            out_shape=[
                jax.ShapeDtypeStruct((T, B, D), jnp.float32),
                jax.ShapeDtypeStruct((T, B, D), jnp.float32),
                jax.ShapeDtypeStruct((BN, D), jnp.float32),
            ],
            scratch_shapes=[
                pltpu.VMEM((B, D), jnp.float32),
                pltpu.VMEM((TBLK, B, D), jnp.float32),
                pltpu.VMEM((BN, D), jnp.bfloat16),
            ],
            compiler_params=pltpu.CompilerParams(
                dimension_semantics=("arbitrary",),
                vmem_limit_bytes=56 * 1024 * 1024,
            ),
            name="elman_seq",
            interpret=interpret,
        )(x_t, Wxp, W_h.T.astype(jnp.bfloat16), W_write.T.astype(jnp.bfloat16),
          H0, h_work_init, b2, bias)

        out_t = pl.pallas_call(
            functools.partial(_gate_body, tblk=GBLK),
            grid=(T // GBLK,),
            in_specs=[
                pl.BlockSpec((GBLK, B, D), lambda i: (i, 0, 0)),
                pl.BlockSpec((GBLK, B, D), lambda i: (i, 0, 0)),
                pl.BlockSpec((GBLK, B, D), lambda i: (i, 0, 0)),
                pl.BlockSpec((D, D), lambda i: (0, 0)),
                pl.BlockSpec((3 * D, D), lambda i: (0, 0)),
            ],
            out_specs=pl.BlockSpec((GBLK, B, D), lambda i: (i, 0, 0)),
            out_shape=jax.ShapeDtypeStruct((T, B, D), jnp.float32),
            compiler_params=pltpu.CompilerParams(
                dimension_semantics=("parallel",),
                vmem_limit_bytes=56 * 1024 * 1024,
            ),
            name="elman_gate",
            interpret=interpret,
        )(x_t, h_all, rv_all, Wz, W_gate.T.astype(jnp.bfloat16))

        return jnp.swapaxes(out_t, 0, 1), tape_f.reshape(B, N, D)

    return forward


_forward = _build()


def kernel(x, h_tape_init, h_work_init, W_h, W_xz, b_h, W_write, W_gate):
    return _forward(x, h_tape_init, h_work_init, W_h, W_xz, b_h,
                    W_write, W_gate)


# speedup vs baseline: 6.7542x; 1.2990x over previous
import functools

import jax
import jax.numpy as jnp
from jax.experimental import pallas as pl
from jax.experimental.pallas import tpu as pltpu


def _seq_body(x_ref, wxp_ref, wh_ref, ww_ref, h0_ref, hw0_ref, b_ref, bias_ref,
              h_all_ref, rv_all_ref, tape_ref, hw_s, xp_s, tape16_s, sr_s,
              *, tblk, scale):
    i = pl.program_id(0)
    B, D = hw0_ref.shape
    BN = tape_ref.shape[0]
    bf16 = jnp.bfloat16
    bias = bias_ref[...]

    @pl.when(i == 0)
    def _init():
        tape_ref[...] = h0_ref[...]
        tape16_s[...] = h0_ref[...].astype(bf16)
        hw_s[...] = hw0_ref[...]
        sr_s[...] = jax.lax.dot_general(
            hw0_ref[...].astype(bf16), h0_ref[...].astype(bf16),
            (((1,), (1,)), ((), ())),
            preferred_element_type=jnp.float32) * scale + bias

    xblk = x_ref[...].reshape(tblk * B, D).astype(bf16)
    xp_s[...] = jnp.dot(xblk, wxp_ref[...],
                        preferred_element_type=jnp.float32).reshape(tblk, B, D)

    ones8 = jnp.ones((B, 8), jnp.float32)

    def step(t, carry):
        h_work, sr = carry
        xp = xp_s[t]
        H16 = tape16_s[...]
        hw16 = h_work.astype(bf16)
        h_lin = jnp.dot(hw16, wh_ref[...], preferred_element_type=jnp.float32)
        e_r = jnp.exp(sr)
        r_rcp = 1.0 / jnp.sum(e_r, axis=-1, keepdims=True)
        rv = jnp.dot(e_r.astype(bf16), H16,
                     preferred_element_type=jnp.float32) * r_rcp
        h_new = jnp.tanh(xp + h_lin + rv + b_ref[...])
        h16 = h_new.astype(bf16)
        wv = jnp.dot(h16, ww_ref[...], preferred_element_type=jnp.float32)
        wv16 = wv.astype(bf16)
        S0 = jax.lax.dot_general(
            h16, H16, (((1,), (1,)), ((), ())),
            preferred_element_type=jnp.float32)
        d = jnp.sum(h_new * wv, axis=-1, keepdims=True)
        s_w = jax.lax.dot_general(
            wv16, H16, (((1,), (1,)), ((), ())),
            preferred_element_type=jnp.float32) * scale + bias
        e_w = jnp.exp(s_w)
        a_w = e_w * (1.0 / jnp.sum(e_w, axis=-1, keepdims=True))
        wa_row = jnp.sum(a_w, axis=0, keepdims=True)
        sr_next = (S0 * (1.0 - wa_row) + d * wa_row) * scale + bias
        wa = jax.lax.dot_general(
            a_w, ones8, (((0,), (0,)), ((), ())),
            preferred_element_type=jnp.float32)[:, :1]
        wv_exp = jnp.broadcast_to(
            wv[:, None, :], (B, BN // B, D)).reshape(BN, D)
        H_new = tape_ref[...] + wa * (wv_exp - tape_ref[...])
        tape_ref[...] = H_new
        tape16_s[...] = H_new.astype(bf16)
        h_all_ref[t] = h_new
        rv_all_ref[t] = rv
        return (h_new, sr_next)

    hw_f, sr_f = jax.lax.fori_loop(0, tblk, step, (hw_s[...], sr_s[...]))
    hw_s[...] = hw_f
    sr_s[...] = sr_f


def _gate_body(x_ref, h_ref, rv_ref, wz_ref, wg_ref, o_ref, *, tblk):
    B, D = x_ref.shape[1], x_ref.shape[2]
    m = tblk * B
    bf16 = jnp.bfloat16
    xb = x_ref[...].reshape(m, D).astype(bf16)
    z = jnp.dot(xb, wz_ref[...], preferred_element_type=jnp.float32)
    h = h_ref[...].reshape(m, D)
    rv = rv_ref[...].reshape(m, D)
    gin = jnp.concatenate(
        [z.astype(bf16), rv.astype(bf16), h.astype(bf16)], axis=1)
    g = jnp.dot(gin, wg_ref[...], preferred_element_type=jnp.float32)
    o_ref[...] = (h * g * jax.nn.sigmoid(g)).reshape(tblk, B, D)


def _build(interpret=False):
    TBLK = 32
    GBLK = 32

    def forward(x, h_tape_init, h_work_init, W_h, W_xz, b_h, W_write, W_gate):
        B, T, D = x.shape
        N = h_tape_init.shape[1]
        BN = B * N
        scale = 1.0 / float(D) ** 0.5

        x_t = jnp.swapaxes(x, 0, 1)
        Wxz_T = W_xz.T.astype(jnp.bfloat16)
        Wxp = Wxz_T[:, :D]
        Wz = Wxz_T[:, D:]
        H0 = h_tape_init.reshape(BN, D)
        b2 = b_h.reshape(1, D)
        cols = jax.lax.broadcasted_iota(jnp.int32, (B, BN), 1) // N
        rows = jax.lax.broadcasted_iota(jnp.int32, (B, BN), 0)
        bias = jnp.where(cols == rows, 0.0, -1e9).astype(jnp.float32)

        h_all, rv_all, tape_f = pl.pallas_call(
            functools.partial(_seq_body, tblk=TBLK, scale=scale),
            grid=(T // TBLK,),
            in_specs=[
                pl.BlockSpec((TBLK, B, D), lambda i: (i, 0, 0)),
                pl.BlockSpec((D, D), lambda i: (0, 0)),
                pl.BlockSpec((D, D), lambda i: (0, 0)),
                pl.BlockSpec((D, D), lambda i: (0, 0)),
                pl.BlockSpec((BN, D), lambda i: (0, 0)),
                pl.BlockSpec((B, D), lambda i: (0, 0)),
                pl.BlockSpec((1, D), lambda i: (0, 0)),
                pl.BlockSpec((B, BN), lambda i: (0, 0)),
            ],
            out_specs=[
                pl.BlockSpec((TBLK, B, D), lambda i: (i, 0, 0)),
                pl.BlockSpec((TBLK, B, D), lambda i: (i, 0, 0)),
                pl.BlockSpec((BN, D), lambda i: (0, 0)),
            ],
            out_shape=[
                jax.ShapeDtypeStruct((T, B, D), jnp.float32),
                jax.ShapeDtypeStruct((T, B, D), jnp.float32),
                jax.ShapeDtypeStruct((BN, D), jnp.float32),
            ],
            scratch_shapes=[
                pltpu.VMEM((B, D), jnp.float32),
                pltpu.VMEM((TBLK, B, D), jnp.float32),
                pltpu.VMEM((BN, D), jnp.bfloat16),
                pltpu.VMEM((B, BN), jnp.float32),
            ],
            compiler_params=pltpu.CompilerParams(
                dimension_semantics=("arbitrary",),
                vmem_limit_bytes=56 * 1024 * 1024,
            ),
            name="elman_seq",
            interpret=interpret,
        )(x_t, Wxp, W_h.T.astype(jnp.bfloat16), W_write.T.astype(jnp.bfloat16),
          H0, h_work_init, b2, bias)

        out_t = pl.pallas_call(
            functools.partial(_gate_body, tblk=GBLK),
            grid=(T // GBLK,),
            in_specs=[
                pl.BlockSpec((GBLK, B, D), lambda i: (i, 0, 0)),
                pl.BlockSpec((GBLK, B, D), lambda i: (i, 0, 0)),
                pl.BlockSpec((GBLK, B, D), lambda i: (i, 0, 0)),
                pl.BlockSpec((D, D), lambda i: (0, 0)),
                pl.BlockSpec((3 * D, D), lambda i: (0, 0)),
            ],
            out_specs=pl.BlockSpec((GBLK, B, D), lambda i: (i, 0, 0)),
            out_shape=jax.ShapeDtypeStruct((T, B, D), jnp.float32),
            compiler_params=pltpu.CompilerParams(
                dimension_semantics=("parallel",),
                vmem_limit_bytes=56 * 1024 * 1024,
            ),
            name="elman_gate",
            interpret=interpret,
        )(x_t, h_all, rv_all, Wz, W_gate.T.astype(jnp.bfloat16))

        return jnp.swapaxes(out_t, 0, 1), tape_f.reshape(B, N, D)

    return forward


_forward = _build()


def kernel(x, h_tape_init, h_work_init, W_h, W_xz, b_h, W_write, W_gate):
    return _forward(x, h_tape_init, h_work_init, W_h, W_xz, b_h,
                    W_write, W_gate)


# speedup vs baseline: 7.8967x; 1.1692x over previous
import functools

import jax
import jax.numpy as jnp
from jax.experimental import pallas as pl
from jax.experimental.pallas import tpu as pltpu


def _seq_body(x_ref, wxp_ref, wh_ref, ww_ref, h0_ref, hw0_ref, b_ref, bias_ref,
              h_all_ref, rv_all_ref, tape_ref, hw_s, xp_s, tape16_s, sr_s,
              *, tblk, scale):
    i = pl.program_id(0)
    B, D = hw0_ref.shape
    BN = tape_ref.shape[0]
    bf16 = jnp.bfloat16
    bias = bias_ref[...]

    @pl.when(i == 0)
    def _init():
        tape_ref[...] = h0_ref[...]
        tape16_s[...] = h0_ref[...].astype(bf16)
        hw_s[...] = hw0_ref[...]
        sr_s[...] = jax.lax.dot_general(
            hw0_ref[...].astype(bf16), h0_ref[...].astype(bf16),
            (((1,), (1,)), ((), ())),
            preferred_element_type=jnp.float32) * scale + bias

    xblk = x_ref[...].reshape(tblk * B, D).astype(bf16)
    xp_s[...] = jnp.dot(xblk, wxp_ref[...],
                        preferred_element_type=jnp.float32).reshape(tblk, B, D)

    eye38 = (jax.lax.broadcasted_iota(jnp.int32, (3, 8), 0)
             == jax.lax.broadcasted_iota(jnp.int32, (3, 8), 1)
             ).astype(jnp.float32)

    def rowsum(v):
        return jnp.sum(v, axis=-1, keepdims=True)

    def pair(k, carry):
        h_work, sr_a = carry
        t0 = 2 * k
        H16 = tape16_s[...]
        hw16 = h_work.astype(bf16)

        h_lin_a = jnp.dot(hw16, wh_ref[...], preferred_element_type=jnp.float32)
        e_a = jnp.exp(sr_a)
        rcp_a = 1.0 / rowsum(e_a)
        rv_a = jnp.dot(e_a.astype(bf16), H16,
                       preferred_element_type=jnp.float32) * rcp_a
        h_b = jnp.tanh(xp_s[t0] + h_lin_a + rv_a + b_ref[...])
        h16b = h_b.astype(bf16)
        wv_a = jnp.dot(h16b, ww_ref[...], preferred_element_type=jnp.float32)
        wv_a16 = wv_a.astype(bf16)
        SS_a = jax.lax.dot_general(
            jnp.concatenate([h16b, wv_a16], axis=0), H16,
            (((1,), (1,)), ((), ())), preferred_element_type=jnp.float32)
        S0a, sw_a_raw = SS_a[:B], SS_a[B:]
        d_a = rowsum(h_b * wv_a)
        e_wa = jnp.exp(sw_a_raw * scale + bias)
        a_wa = e_wa * (1.0 / rowsum(e_wa))
        wa1 = jnp.sum(a_wa, axis=0, keepdims=True)
        nw1 = 1.0 - wa1
        sr_b = (S0a * nw1 + d_a * wa1) * scale + bias

        h_lin_b = jnp.dot(h16b, wh_ref[...], preferred_element_type=jnp.float32)
        e_b = jnp.exp(sr_b)
        rcp_b = 1.0 / rowsum(e_b)
        rv_b = (jnp.dot((e_b * nw1).astype(bf16), H16,
                        preferred_element_type=jnp.float32)
                + rowsum(e_b * wa1) * wv_a) * rcp_b
        h_c = jnp.tanh(xp_s[t0 + 1] + h_lin_b + rv_b + b_ref[...])
        h16c = h_c.astype(bf16)
        wv_b = jnp.dot(h16c, ww_ref[...], preferred_element_type=jnp.float32)
        wv_b16 = wv_b.astype(bf16)
        SS_b = jax.lax.dot_general(
            jnp.concatenate([h16c, wv_b16], axis=0), H16,
            (((1,), (1,)), ((), ())), preferred_element_type=jnp.float32)
        S0b, sw_b0 = SS_b[:B], SS_b[B:]
        dw = rowsum(wv_b * wv_a)
        s_w_b = (sw_b0 * nw1 + dw * wa1) * scale + bias
        e_wb = jnp.exp(s_w_b)
        a_wb = e_wb * (1.0 / rowsum(e_wb))
        wa2 = jnp.sum(a_wb, axis=0, keepdims=True)
        nw2 = 1.0 - wa2
        d_ba = rowsum(h_c * wv_a)
        d_bb = rowsum(h_c * wv_b)
        sr_c = ((S0b * nw1 + d_ba * wa1) * nw2 + d_bb * wa2) * scale + bias

        c_rows = jnp.concatenate([nw1 * nw2, wa1 * nw2, wa2], axis=0)
        Ct = jax.lax.dot_general(
            c_rows, eye38, (((0,), (0,)), ((), ())),
            preferred_element_type=jnp.float32)
        wva_exp = jnp.broadcast_to(
            wv_a[:, None, :], (B, BN // B, D)).reshape(BN, D)
        wvb_exp = jnp.broadcast_to(
            wv_b[:, None, :], (B, BN // B, D)).reshape(BN, D)
        H_new = (tape_ref[...] * Ct[:, 0:1] + wva_exp * Ct[:, 1:2]
                 + wvb_exp * Ct[:, 2:3])
        tape_ref[...] = H_new
        tape16_s[...] = H_new.astype(bf16)
        h_all_ref[t0] = h_b
        h_all_ref[t0 + 1] = h_c
        rv_all_ref[t0] = rv_a
        rv_all_ref[t0 + 1] = rv_b
        return (h_c, sr_c)

    hw_f, sr_f = jax.lax.fori_loop(0, tblk // 2, pair, (hw_s[...], sr_s[...]))
    hw_s[...] = hw_f
    sr_s[...] = sr_f


def _gate_body(x_ref, h_ref, rv_ref, wz_ref, wg_ref, o_ref, *, tblk):
    B, D = x_ref.shape[1], x_ref.shape[2]
    m = tblk * B
    bf16 = jnp.bfloat16
    xb = x_ref[...].reshape(m, D).astype(bf16)
    z = jnp.dot(xb, wz_ref[...], preferred_element_type=jnp.float32)
    h = h_ref[...].reshape(m, D)
    rv = rv_ref[...].reshape(m, D)
    gin = jnp.concatenate(
        [z.astype(bf16), rv.astype(bf16), h.astype(bf16)], axis=1)
    g = jnp.dot(gin, wg_ref[...], preferred_element_type=jnp.float32)
    o_ref[...] = (h * g * jax.nn.sigmoid(g)).reshape(tblk, B, D)


def _build(interpret=False):
    TBLK = 32
    GBLK = 32

    def forward(x, h_tape_init, h_work_init, W_h, W_xz, b_h, W_write, W_gate):
        B, T, D = x.shape
        N = h_tape_init.shape[1]
        BN = B * N
        scale = 1.0 / float(D) ** 0.5

        x_t = jnp.swapaxes(x, 0, 1)
        Wxz_T = W_xz.T.astype(jnp.bfloat16)
        Wxp = Wxz_T[:, :D]
        Wz = Wxz_T[:, D:]
        H0 = h_tape_init.reshape(BN, D)
        b2 = b_h.reshape(1, D)
        cols = jax.lax.broadcasted_iota(jnp.int32, (B, BN), 1) // N
        rows = jax.lax.broadcasted_iota(jnp.int32, (B, BN), 0)
        bias = jnp.where(cols == rows, 0.0, -1e9).astype(jnp.float32)

        h_all, rv_all, tape_f = pl.pallas_call(
            functools.partial(_seq_body, tblk=TBLK, scale=scale),
            grid=(T // TBLK,),
            in_specs=[
                pl.BlockSpec((TBLK, B, D), lambda i: (i, 0, 0)),
                pl.BlockSpec((D, D), lambda i: (0, 0)),
                pl.BlockSpec((D, D), lambda i: (0, 0)),
                pl.BlockSpec((D, D), lambda i: (0, 0)),
                pl.BlockSpec((BN, D), lambda i: (0, 0)),
                pl.BlockSpec((B, D), lambda i: (0, 0)),
                pl.BlockSpec((1, D), lambda i: (0, 0)),
                pl.BlockSpec((B, BN), lambda i: (0, 0)),
            ],
            out_specs=[
                pl.BlockSpec((TBLK, B, D), lambda i: (i, 0, 0)),
                pl.BlockSpec((TBLK, B, D), lambda i: (i, 0, 0)),
                pl.BlockSpec((BN, D), lambda i: (0, 0)),
            ],
            out_shape=[
                jax.ShapeDtypeStruct((T, B, D), jnp.float32),
                jax.ShapeDtypeStruct((T, B, D), jnp.float32),
                jax.ShapeDtypeStruct((BN, D), jnp.float32),
            ],
            scratch_shapes=[
                pltpu.VMEM((B, D), jnp.float32),
                pltpu.VMEM((TBLK, B, D), jnp.float32),
                pltpu.VMEM((BN, D), jnp.bfloat16),
                pltpu.VMEM((B, BN), jnp.float32),
            ],
            compiler_params=pltpu.CompilerParams(
                dimension_semantics=("arbitrary",),
                vmem_limit_bytes=56 * 1024 * 1024,
            ),
            name="elman_seq",
            interpret=interpret,
        )(x_t, Wxp, W_h.T.astype(jnp.bfloat16), W_write.T.astype(jnp.bfloat16),
          H0, h_work_init, b2, bias)

        out_t = pl.pallas_call(
            functools.partial(_gate_body, tblk=GBLK),
            grid=(T // GBLK,),
            in_specs=[
                pl.BlockSpec((GBLK, B, D), lambda i: (i, 0, 0)),
                pl.BlockSpec((GBLK, B, D), lambda i: (i, 0, 0)),
                pl.BlockSpec((GBLK, B, D), lambda i: (i, 0, 0)),
                pl.BlockSpec((D, D), lambda i: (0, 0)),
                pl.BlockSpec((3 * D, D), lambda i: (0, 0)),
            ],
            out_specs=pl.BlockSpec((GBLK, B, D), lambda i: (i, 0, 0)),
            out_shape=jax.ShapeDtypeStruct((T, B, D), jnp.float32),
            compiler_params=pltpu.CompilerParams(
                dimension_semantics=("parallel",),
                vmem_limit_bytes=56 * 1024 * 1024,
            ),
            name="elman_gate",
            interpret=interpret,
        )(x_t, h_all, rv_all, Wz, W_gate.T.astype(jnp.bfloat16))

        return jnp.swapaxes(out_t, 0, 1), tape_f.reshape(B, N, D)

    return forward


_forward = _build()


def kernel(x, h_tape_init, h_work_init, W_h, W_xz, b_h, W_write, W_gate):
    return _forward(x, h_tape_init, h_work_init, W_h, W_xz, b_h,
                    W_write, W_gate)


# speedup vs baseline: 7.9202x; 1.0030x over previous
import functools

import jax
import jax.numpy as jnp
from jax.experimental import pallas as pl
from jax.experimental.pallas import tpu as pltpu


def _seq_body(x_ref, wxp_ref, wh_ref, ww_ref, h0_ref, hw0_ref, b_ref, bias_ref,
              h_all_ref, rv_all_ref, tape_out_ref, hw_s, xp_s, tape_s,
              tape16_s, sr_s, p_s, u_s, *, tblk, scale, nblocks):
    i = pl.program_id(0)
    B, D = hw0_ref.shape
    BN = tape_out_ref.shape[0]
    bf16 = jnp.bfloat16
    bias = bias_ref[...]
    npairs = tblk // 2

    @pl.when(i == 0)
    def _init():
        tape_s[0] = h0_ref[...]
        tape16_s[0] = h0_ref[...].astype(bf16)
        hw_s[...] = hw0_ref[...]
        sr_s[...] = jax.lax.dot_general(
            hw0_ref[...].astype(bf16), h0_ref[...].astype(bf16),
            (((1,), (1,)), ((), ())),
            preferred_element_type=jnp.float32) * scale + bias
        col = jax.lax.broadcasted_iota(jnp.int32, (8, BN), 0)
        p_s[...] = (col == 0).astype(jnp.float32)
        u_s[...] = jnp.zeros_like(u_s)

    xblk = x_ref[...].reshape(tblk * B, D).astype(bf16)
    xp_s[...] = jnp.dot(xblk, wxp_ref[...],
                        preferred_element_type=jnp.float32).reshape(tblk, B, D)

    eye38 = (jax.lax.broadcasted_iota(jnp.int32, (3, 8), 0)
             == jax.lax.broadcasted_iota(jnp.int32, (3, 8), 1)
             ).astype(jnp.float32)

    def rowsum(v):
        return jnp.sum(v, axis=-1, keepdims=True)

    def expand(v):
        return jnp.broadcast_to(v[:, None, :], (B, BN // B, D)).reshape(BN, D)

    def apply_update(H_old, p_rows, u_a, u_b):
        Ct = jax.lax.dot_general(
            p_rows, eye38, (((0,), (0,)), ((), ())),
            preferred_element_type=jnp.float32)
        return (H_old * Ct[:, 0:1] + expand(u_a) * Ct[:, 1:2]
                + expand(u_b) * Ct[:, 2:3])

    def pair(k, carry):
        h_work, sr_a = carry
        t0 = 2 * k
        rb = k % 2
        wb = (k + 1) % 2
        p_rows = p_s[0:3, :]
        p0, p1, p2 = p_rows[0:1], p_rows[1:2], p_rows[2:3]
        u_a = u_s[0]
        u_b = u_s[1]
        H_cur = apply_update(tape_s[rb], p_rows, u_a, u_b)
        tape_s[wb] = H_cur
        tape16_s[wb] = H_cur.astype(bf16)

        H16 = tape16_s[rb]
        hw16 = h_work.astype(bf16)

        h_lin_a = jnp.dot(hw16, wh_ref[...], preferred_element_type=jnp.float32)
        e_a = jnp.exp(sr_a)
        rcp_a = 1.0 / rowsum(e_a)
        rv_a = (jnp.dot((e_a * p0).astype(bf16), H16,
                        preferred_element_type=jnp.float32)
                + rowsum(e_a * p1) * u_a
                + rowsum(e_a * p2) * u_b) * rcp_a
        h_b = jnp.tanh(xp_s[t0] + h_lin_a + rv_a + b_ref[...])
        h16b = h_b.astype(bf16)
        wv_a = jnp.dot(h16b, ww_ref[...], preferred_element_type=jnp.float32)
        wv_a16 = wv_a.astype(bf16)
        SS_a = jax.lax.dot_general(
            jnp.concatenate([h16b, wv_a16], axis=0), H16,
            (((1,), (1,)), ((), ())), preferred_element_type=jnp.float32)
        S0a = (SS_a[:B] * p0 + rowsum(h_b * u_a) * p1
               + rowsum(h_b * u_b) * p2)
        sw_a = (SS_a[B:] * p0 + rowsum(wv_a * u_a) * p1
                + rowsum(wv_a * u_b) * p2)
        d_a = rowsum(h_b * wv_a)
        e_wa = jnp.exp(sw_a * scale + bias)
        a_wa = e_wa * (1.0 / rowsum(e_wa))
        wa1 = jnp.sum(a_wa, axis=0, keepdims=True)
        nw1 = 1.0 - wa1
        sr_b = (S0a * nw1 + d_a * wa1) * scale + bias

        h_lin_b = jnp.dot(h16b, wh_ref[...], preferred_element_type=jnp.float32)
        e_b = jnp.exp(sr_b)
        rcp_b = 1.0 / rowsum(e_b)
        q0 = p0 * nw1
        rv_b = (jnp.dot((e_b * q0).astype(bf16), H16,
                        preferred_element_type=jnp.float32)
                + rowsum(e_b * (p1 * nw1)) * u_a
                + rowsum(e_b * (p2 * nw1)) * u_b
                + rowsum(e_b * wa1) * wv_a) * rcp_b
        h_c = jnp.tanh(xp_s[t0 + 1] + h_lin_b + rv_b + b_ref[...])
        h16c = h_c.astype(bf16)
        wv_b = jnp.dot(h16c, ww_ref[...], preferred_element_type=jnp.float32)
        wv_b16 = wv_b.astype(bf16)
        SS_b = jax.lax.dot_general(
            jnp.concatenate([h16c, wv_b16], axis=0), H16,
            (((1,), (1,)), ((), ())), preferred_element_type=jnp.float32)
        S0b = ((SS_b[:B] * p0 + rowsum(h_c * u_a) * p1
                + rowsum(h_c * u_b) * p2) * nw1
               + rowsum(h_c * wv_a) * wa1)
        sw_b = ((SS_b[B:] * p0 + rowsum(wv_b * u_a) * p1
                 + rowsum(wv_b * u_b) * p2) * nw1
                + rowsum(wv_b * wv_a) * wa1)
        e_wb = jnp.exp(sw_b * scale + bias)
        a_wb = e_wb * (1.0 / rowsum(e_wb))
        wa2 = jnp.sum(a_wb, axis=0, keepdims=True)
        nw2 = 1.0 - wa2
        d_bb = rowsum(h_c * wv_b)
        sr_c = (S0b * nw2 + d_bb * wa2) * scale + bias

        p_s[0:3, :] = jnp.concatenate([nw1 * nw2, wa1 * nw2, wa2], axis=0)
        u_s[0] = wv_a
        u_s[1] = wv_b

        @pl.when((i == nblocks - 1) & (k == npairs - 1))
        def _final():
            tape_out_ref[...] = apply_update(
                H_cur, jnp.concatenate([nw1 * nw2, wa1 * nw2, wa2], axis=0),
                wv_a, wv_b)

        h_all_ref[t0] = h_b
        h_all_ref[t0 + 1] = h_c
        rv_all_ref[t0] = rv_a
        rv_all_ref[t0 + 1] = rv_b
        return (h_c, sr_c)

    hw_f, sr_f = jax.lax.fori_loop(0, npairs, pair, (hw_s[...], sr_s[...]))
    hw_s[...] = hw_f
    sr_s[...] = sr_f


def _gate_body(x_ref, h_ref, rv_ref, wz_ref, wg_ref, o_ref, *, tblk):
    B, D = x_ref.shape[1], x_ref.shape[2]
    m = tblk * B
    bf16 = jnp.bfloat16
    xb = x_ref[...].reshape(m, D).astype(bf16)
    z = jnp.dot(xb, wz_ref[...], preferred_element_type=jnp.float32)
    h = h_ref[...].reshape(m, D)
    rv = rv_ref[...].reshape(m, D)
    gin = jnp.concatenate(
        [z.astype(bf16), rv.astype(bf16), h.astype(bf16)], axis=1)
    g = jnp.dot(gin, wg_ref[...], preferred_element_type=jnp.float32)
    o_ref[...] = (h * g * jax.nn.sigmoid(g)).reshape(tblk, B, D)


def _build(interpret=False):
    TBLK = 32
    GBLK = 32

    def forward(x, h_tape_init, h_work_init, W_h, W_xz, b_h, W_write, W_gate):
        B, T, D = x.shape
        N = h_tape_init.shape[1]
        BN = B * N
        scale = 1.0 / float(D) ** 0.5

        x_t = jnp.swapaxes(x, 0, 1)
        Wxz_T = W_xz.T.astype(jnp.bfloat16)
        Wxp = Wxz_T[:, :D]
        Wz = Wxz_T[:, D:]
        H0 = h_tape_init.reshape(BN, D)
        b2 = b_h.reshape(1, D)
        cols = jax.lax.broadcasted_iota(jnp.int32, (B, BN), 1) // N
        rows = jax.lax.broadcasted_iota(jnp.int32, (B, BN), 0)
        bias = jnp.where(cols == rows, 0.0, -1e9).astype(jnp.float32)

        h_all, rv_all, tape_f = pl.pallas_call(
            functools.partial(_seq_body, tblk=TBLK, scale=scale,
                              nblocks=T // TBLK),
            grid=(T // TBLK,),
            in_specs=[
                pl.BlockSpec((TBLK, B, D), lambda i: (i, 0, 0)),
                pl.BlockSpec((D, D), lambda i: (0, 0)),
                pl.BlockSpec((D, D), lambda i: (0, 0)),
                pl.BlockSpec((D, D), lambda i: (0, 0)),
                pl.BlockSpec((BN, D), lambda i: (0, 0)),
                pl.BlockSpec((B, D), lambda i: (0, 0)),
                pl.BlockSpec((1, D), lambda i: (0, 0)),
                pl.BlockSpec((B, BN), lambda i: (0, 0)),
            ],
            out_specs=[
                pl.BlockSpec((TBLK, B, D), lambda i: (i, 0, 0)),
                pl.BlockSpec((TBLK, B, D), lambda i: (i, 0, 0)),
                pl.BlockSpec((BN, D), lambda i: (0, 0)),
            ],
            out_shape=[
                jax.ShapeDtypeStruct((T, B, D), jnp.float32),
                jax.ShapeDtypeStruct((T, B, D), jnp.float32),
                jax.ShapeDtypeStruct((BN, D), jnp.float32),
            ],
            scratch_shapes=[
                pltpu.VMEM((B, D), jnp.float32),
                pltpu.VMEM((TBLK, B, D), jnp.float32),
                pltpu.VMEM((2, BN, D), jnp.float32),
                pltpu.VMEM((2, BN, D), jnp.bfloat16),
                pltpu.VMEM((B, BN), jnp.float32),
                pltpu.VMEM((8, BN), jnp.float32),
                pltpu.VMEM((2, B, D), jnp.float32),
            ],
            compiler_params=pltpu.CompilerParams(
                dimension_semantics=("arbitrary",),
                vmem_limit_bytes=56 * 1024 * 1024,
            ),
            name="elman_seq",
            interpret=interpret,
        )(x_t, Wxp, W_h.T.astype(jnp.bfloat16), W_write.T.astype(jnp.bfloat16),
          H0, h_work_init, b2, bias)

        out_t = pl.pallas_call(
            functools.partial(_gate_body, tblk=GBLK),
            grid=(T // GBLK,),
            in_specs=[
                pl.BlockSpec((GBLK, B, D), lambda i: (i, 0, 0)),
                pl.BlockSpec((GBLK, B, D), lambda i: (i, 0, 0)),
                pl.BlockSpec((GBLK, B, D), lambda i: (i, 0, 0)),
                pl.BlockSpec((D, D), lambda i: (0, 0)),
                pl.BlockSpec((3 * D, D), lambda i: (0, 0)),
            ],
            out_specs=pl.BlockSpec((GBLK, B, D), lambda i: (i, 0, 0)),
            out_shape=jax.ShapeDtypeStruct((T, B, D), jnp.float32),
            compiler_params=pltpu.CompilerParams(
                dimension_semantics=("parallel",),
                vmem_limit_bytes=56 * 1024 * 1024,
            ),
            name="elman_gate",
            interpret=interpret,
        )(x_t, h_all, rv_all, Wz, W_gate.T.astype(jnp.bfloat16))

        return jnp.swapaxes(out_t, 0, 1), tape_f.reshape(B, N, D)

    return forward


_forward = _build()


def kernel(x, h_tape_init, h_work_init, W_h, W_xz, b_h, W_write, W_gate):
    return _forward(x, h_tape_init, h_work_init, W_h, W_xz, b_h,
                    W_write, W_gate)


# speedup vs baseline: 7.9269x; 1.0008x over previous
import functools

import jax
import jax.numpy as jnp
from jax.experimental import pallas as pl
from jax.experimental.pallas import tpu as pltpu


def _seq_body(x_ref, wxp_ref, wh_ref, ww_ref, h0_ref, hw0_ref, b_ref, bias_ref,
              h_all_ref, rv_all_ref, tape_out_ref, hw_s, xp_s, tape_s,
              tape16_s, sr_s, p_s, u_s, *, tblk, scale, nblocks):
    i = pl.program_id(0)
    B, D = hw0_ref.shape
    BN = tape_out_ref.shape[0]
    bf16 = jnp.bfloat16
    bias = bias_ref[...]
    npairs = tblk // 2

    @pl.when(i == 0)
    def _init():
        tape_s[0] = h0_ref[...]
        tape16_s[0] = h0_ref[...].astype(bf16)
        hw_s[...] = hw0_ref[...]
        sr_s[...] = jax.lax.dot_general(
            hw0_ref[...].astype(bf16), h0_ref[...].astype(bf16),
            (((1,), (1,)), ((), ())),
            preferred_element_type=jnp.float32) * scale + bias
        col = jax.lax.broadcasted_iota(jnp.int32, (8, BN), 0)
        p_s[...] = (col == 0).astype(jnp.float32)
        u_s[...] = jnp.zeros_like(u_s)

    xblk = x_ref[...].reshape(tblk * B, D).astype(bf16)
    xp_s[...] = jnp.dot(xblk, wxp_ref[...],
                        preferred_element_type=jnp.float32).reshape(tblk, B, D)

    eye38 = (jax.lax.broadcasted_iota(jnp.int32, (3, 8), 0)
             == jax.lax.broadcasted_iota(jnp.int32, (3, 8), 1)
             ).astype(jnp.float32)

    def rowsum(v):
        return jnp.sum(v, axis=-1, keepdims=True)

    def expand(v):
        return jnp.broadcast_to(v[:, None, :], (B, BN // B, D)).reshape(BN, D)

    def apply_update(H_old, p_rows, u_a, u_b):
        Ct = jax.lax.dot_general(
            p_rows, eye38, (((0,), (0,)), ((), ())),
            preferred_element_type=jnp.float32)
        return (H_old * Ct[:, 0:1] + expand(u_a) * Ct[:, 1:2]
                + expand(u_b) * Ct[:, 2:3])

    def pair(k, carry):
        h_work, sr_a = carry
        t0 = 2 * k
        rb = k % 2
        wb = (k + 1) % 2
        p_rows = p_s[0:3, :]
        p0, p1, p2 = p_rows[0:1], p_rows[1:2], p_rows[2:3]
        u_a = u_s[0]
        u_b = u_s[1]
        H_cur = apply_update(tape_s[rb], p_rows, u_a, u_b)
        tape_s[wb] = H_cur
        tape16_s[wb] = H_cur.astype(bf16)

        H16 = tape16_s[rb]
        hw16 = h_work.astype(bf16)

        h_lin_a = jnp.dot(hw16, wh_ref[...], preferred_element_type=jnp.float32)
        e_a = jnp.exp(sr_a)
        rcp_a = 1.0 / rowsum(e_a)
        rv_a = (jnp.dot((e_a * p0).astype(bf16), H16,
                        preferred_element_type=jnp.float32)
                + rowsum(e_a * p1) * u_a
                + rowsum(e_a * p2) * u_b) * rcp_a
        h_b = jnp.tanh(xp_s[t0] + h_lin_a + rv_a + b_ref[...])
        h16b = h_b.astype(bf16)
        wv_a = jnp.dot(h16b, ww_ref[...], preferred_element_type=jnp.float32)
        wv_a16 = wv_a.astype(bf16)
        SS_a = jax.lax.dot_general(
            jnp.concatenate([h16b, wv_a16], axis=0), H16,
            (((1,), (1,)), ((), ())), preferred_element_type=jnp.float32)
        S0a = (SS_a[:B] * p0 + rowsum(h_b * u_a) * p1
               + rowsum(h_b * u_b) * p2)
        sw_a = (SS_a[B:] * p0 + rowsum(wv_a * u_a) * p1
                + rowsum(wv_a * u_b) * p2)
        d_a = rowsum(h_b * wv_a)
        e_wa = jnp.exp(sw_a * scale + bias)
        a_wa = e_wa * (1.0 / rowsum(e_wa))
        wa1 = jnp.sum(a_wa, axis=0, keepdims=True)
        nw1 = 1.0 - wa1
        sr_b = (S0a * nw1 + d_a * wa1) * scale + bias

        h_lin_b = jnp.dot(h16b, wh_ref[...], preferred_element_type=jnp.float32)
        e_b = jnp.exp(sr_b)
        rcp_b = 1.0 / rowsum(e_b)
        q0 = p0 * nw1
        rv_b = (jnp.dot((e_b * q0).astype(bf16), H16,
                        preferred_element_type=jnp.float32)
                + rowsum(e_b * (p1 * nw1)) * u_a
                + rowsum(e_b * (p2 * nw1)) * u_b
                + rowsum(e_b * wa1) * wv_a) * rcp_b
        h_c = jnp.tanh(xp_s[t0 + 1] + h_lin_b + rv_b + b_ref[...])
        h16c = h_c.astype(bf16)
        wv_b = jnp.dot(h16c, ww_ref[...], preferred_element_type=jnp.float32)
        wv_b16 = wv_b.astype(bf16)
        SS_b = jax.lax.dot_general(
            jnp.concatenate([h16c, wv_b16], axis=0), H16,
            (((1,), (1,)), ((), ())), preferred_element_type=jnp.float32)
        S0b = ((SS_b[:B] * p0 + rowsum(h_c * u_a) * p1
                + rowsum(h_c * u_b) * p2) * nw1
               + rowsum(h_c * wv_a) * wa1)
        sw_b = ((SS_b[B:] * p0 + rowsum(wv_b * u_a) * p1
                 + rowsum(wv_b * u_b) * p2) * nw1
                + rowsum(wv_b * wv_a) * wa1)
        e_wb = jnp.exp(sw_b * scale + bias)
        a_wb = e_wb * (1.0 / rowsum(e_wb))
        wa2 = jnp.sum(a_wb, axis=0, keepdims=True)
        nw2 = 1.0 - wa2
        d_bb = rowsum(h_c * wv_b)
        sr_c = (S0b * nw2 + d_bb * wa2) * scale + bias

        p_s[0:3, :] = jnp.concatenate([nw1 * nw2, wa1 * nw2, wa2], axis=0)
        u_s[0] = wv_a
        u_s[1] = wv_b

        h_all_ref[t0] = h_b
        h_all_ref[t0 + 1] = h_c
        rv_all_ref[t0] = rv_a.astype(rv_all_ref.dtype)
        rv_all_ref[t0 + 1] = rv_b.astype(rv_all_ref.dtype)
        return (h_c, sr_c)

    hw_f, sr_f = jax.lax.fori_loop(0, npairs, pair, (hw_s[...], sr_s[...]))
    hw_s[...] = hw_f
    sr_s[...] = sr_f

    @pl.when(i == nblocks - 1)
    def _final():
        tape_out_ref[...] = apply_update(
            tape_s[0], p_s[0:3, :], u_s[0], u_s[1])


def _gate_body(x_ref, h_ref, rv_ref, wz_ref, wg_ref, o_ref, *, tblk):
    B, D = x_ref.shape[1], x_ref.shape[2]
    m = tblk * B
    bf16 = jnp.bfloat16
    xb = x_ref[...].reshape(m, D).astype(bf16)
    z = jnp.dot(xb, wz_ref[...], preferred_element_type=jnp.float32)
    h = h_ref[...].reshape(m, D)
    rv = rv_ref[...].reshape(m, D)
    gin = jnp.concatenate(
        [z.astype(bf16), rv.astype(bf16), h.astype(bf16)], axis=1)
    g = jnp.dot(gin, wg_ref[...], preferred_element_type=jnp.float32)
    o_ref[...] = (h * g * jax.nn.sigmoid(g)).reshape(tblk, B, D)


def _build(interpret=False):
    TBLK = 32
    GBLK = 32

    def forward(x, h_tape_init, h_work_init, W_h, W_xz, b_h, W_write, W_gate):
        B, T, D = x.shape
        N = h_tape_init.shape[1]
        BN = B * N
        scale = 1.0 / float(D) ** 0.5

        x_t = jnp.swapaxes(x, 0, 1).astype(jnp.bfloat16)
        Wxz_T = W_xz.T.astype(jnp.bfloat16)
        Wxp = Wxz_T[:, :D]
        Wz = Wxz_T[:, D:]
        H0 = h_tape_init.reshape(BN, D)
        b2 = b_h.reshape(1, D)
        cols = jax.lax.broadcasted_iota(jnp.int32, (B, BN), 1) // N
        rows = jax.lax.broadcasted_iota(jnp.int32, (B, BN), 0)
        bias = jnp.where(cols == rows, 0.0, -1e9).astype(jnp.float32)

        h_all, rv_all, tape_f = pl.pallas_call(
            functools.partial(_seq_body, tblk=TBLK, scale=scale,
                              nblocks=T // TBLK),
            grid=(T // TBLK,),
            in_specs=[
                pl.BlockSpec((TBLK, B, D), lambda i: (i, 0, 0)),
                pl.BlockSpec((D, D), lambda i: (0, 0)),
                pl.BlockSpec((D, D), lambda i: (0, 0)),
                pl.BlockSpec((D, D), lambda i: (0, 0)),
                pl.BlockSpec((BN, D), lambda i: (0, 0)),
                pl.BlockSpec((B, D), lambda i: (0, 0)),
                pl.BlockSpec((1, D), lambda i: (0, 0)),
                pl.BlockSpec((B, BN), lambda i: (0, 0)),
            ],
            out_specs=[
                pl.BlockSpec((TBLK, B, D), lambda i: (i, 0, 0)),
                pl.BlockSpec((TBLK, B, D), lambda i: (i, 0, 0)),
                pl.BlockSpec((BN, D), lambda i: (0, 0)),
            ],
            out_shape=[
                jax.ShapeDtypeStruct((T, B, D), jnp.float32),
                jax.ShapeDtypeStruct((T, B, D), jnp.bfloat16),
                jax.ShapeDtypeStruct((BN, D), jnp.float32),
            ],
            scratch_shapes=[
                pltpu.VMEM((B, D), jnp.float32),
                pltpu.VMEM((TBLK, B, D), jnp.float32),
                pltpu.VMEM((2, BN, D), jnp.float32),
                pltpu.VMEM((2, BN, D), jnp.bfloat16),
                pltpu.VMEM((B, BN), jnp.float32),
                pltpu.VMEM((8, BN), jnp.float32),
                pltpu.VMEM((2, B, D), jnp.float32),
            ],
            compiler_params=pltpu.CompilerParams(
                dimension_semantics=("arbitrary",),
                vmem_limit_bytes=56 * 1024 * 1024,
            ),
            name="elman_seq",
            interpret=interpret,
        )(x_t, Wxp, W_h.T.astype(jnp.bfloat16), W_write.T.astype(jnp.bfloat16),
          H0, h_work_init, b2, bias)

        out_t = pl.pallas_call(
            functools.partial(_gate_body, tblk=GBLK),
            grid=(T // GBLK,),
            in_specs=[
                pl.BlockSpec((GBLK, B, D), lambda i: (i, 0, 0)),
                pl.BlockSpec((GBLK, B, D), lambda i: (i, 0, 0)),
                pl.BlockSpec((GBLK, B, D), lambda i: (i, 0, 0)),
                pl.BlockSpec((D, D), lambda i: (0, 0)),
                pl.BlockSpec((3 * D, D), lambda i: (0, 0)),
            ],
            out_specs=pl.BlockSpec((GBLK, B, D), lambda i: (i, 0, 0)),
            out_shape=jax.ShapeDtypeStruct((T, B, D), jnp.float32),
            compiler_params=pltpu.CompilerParams(
                dimension_semantics=("parallel",),
                vmem_limit_bytes=56 * 1024 * 1024,
            ),
            name="elman_gate",
            interpret=interpret,
        )(x_t, h_all, rv_all, Wz, W_gate.T.astype(jnp.bfloat16))

        return jnp.swapaxes(out_t, 0, 1), tape_f.reshape(B, N, D)

    return forward


_forward = _build()


def kernel(x, h_tape_init, h_work_init, W_h, W_xz, b_h, W_write, W_gate):
    return _forward(x, h_tape_init, h_work_init, W_h, W_xz, b_h,
                    W_write, W_gate)
